# baseline (speedup 1.0000x reference)
import functools

import jax
import jax.numpy as jnp
from jax import lax
from jax.experimental import pallas as pl
from jax.experimental.pallas import tpu as pltpu
from jax.experimental.pallas import tpu_sc as plsc

N = 10000
NP = 10240
D = 128
NNZ = 330000
NC = 2
NS = 16
WHP = 128
B = 128
ROWS_PER_TILE = NP // NS
EB_TILE = -(-NNZ // (NS * B))
E_TILE = EB_TILE * B
NTOT = E_TILE * NS
NK = WHP // 16


def _lane_bcast(vv, e):
    return lax.gather(
        vv, jnp.full((16, 1), e, jnp.int32),
        lax.GatherDimensionNumbers(offset_dims=(), collapsed_slice_dims=(0,),
                                   start_index_map=(0,)),
        (1,), mode=lax.GatherScatterMode.PROMISE_IN_BOUNDS)


def _spmm_body(x_hbm, cols_hbm, rows_hbm, vals_hbm, y_hbm,
               acc, cbuf, rbuf, vbuf, gath, gsem0, gsem1, csem0, csem1,
               rsem0, rsem1, vsem0, vsem1, ssem0, ssem1):
    c = lax.axis_index("c")
    s = lax.axis_index("s")
    row0 = s * ROWS_PER_TILE
    coff = c * NP
    gsems = (gsem0, gsem1)
    csems = (csem0, csem1)
    rsems = (rsem0, rsem1)
    vsems = (vsem0, vsem1)
    ssems = (ssem0, ssem1)
    NT = EB_TILE // 2

    zero16 = jnp.zeros((16,), jnp.float32)

    @pl.loop(0, B)
    def _zero_gath(j):
        for k in range(NK):
            gath[0, j, pl.ds(k * 16, 16)] = zero16

    @pl.loop(0, ROWS_PER_TILE // B)
    def _zero_acc(i):
        pltpu.sync_copy(gath.at[0], acc.at[pl.ds(row0 + i * B, B)])

    plsc.subcore_barrier()

    def cdesc(b, u):
        return pltpu.make_async_copy(cols_hbm.at[s, b], cbuf.at[u], csems[u])

    def rdesc(b, u):
        return pltpu.make_async_copy(rows_hbm.at[s, b], rbuf.at[u], rsems[u])

    def vdesc(b, u):
        return pltpu.make_async_copy(vals_hbm.at[s, b], vbuf.at[u], vsems[u])

    def gdesc(u):
        return pltpu.make_async_copy(x_hbm.at[cbuf.at[u, 0]], gath.at[u],
                                     gsems[u])

    def add_coff(u):
        for k in range(B // 16):
            sl = (u, 0, pl.ds(k * 16, 16))
            cbuf[sl] = cbuf[sl] + coff

    def sdesc(u):
        return pltpu.make_async_copy(gath.at[u], acc.at[rbuf.at[u, 0]],
                                     ssems[u])

    def scale(u):
        @pl.loop(0, B // 16)
        def _scale(g):
            vv = vbuf[u, 0, pl.ds(g * 16, 16)]
            for e in range(16):
                v = _lane_bcast(vv, e)
                j = g * 16 + e
                for k in range(NK):
                    sl = (u, j, pl.ds(k * 16, 16))
                    gath[sl] = gath[sl] * v

    pltpu.sync_copy(cols_hbm.at[s, 0], cbuf.at[0])
    rdesc(0, 0).start()
    vdesc(0, 0).start()
    add_coff(0)
    gdesc(0).start()
    cdesc(1, 1).start()
    rdesc(1, 1).start()
    vdesc(1, 1).start()

    def process(b, u):
        rdesc(b, u).wait()
        vdesc(b, u).wait()
        scale(u)
        pltpu.sync_copy(gath.at[u], acc.at[rbuf.at[u, 0]], add=True)

    @pl.loop(0, NT)
    def _pair(t):
        b0 = t * 2
        not_last = t < NT - 1

        cdesc(b0 + 1, 1).wait()
        add_coff(1)
        gdesc(1).start()
        gdesc(0).wait()

        @pl.when(not_last)
        def _():
            cdesc(b0 + 2, 0).start()
        process(b0, 0)

        @pl.when(not_last)
        def _():
            rdesc(b0 + 2, 0).start()
            vdesc(b0 + 2, 0).start()

        @pl.when(not_last)
        def _():
            cdesc(b0 + 2, 0).wait()
            add_coff(0)
            gdesc(0).start()
        gdesc(1).wait()

        process(b0 + 1, 1)

        @pl.when(not_last)
        def _():
            cdesc(b0 + 3, 1).start()
            rdesc(b0 + 3, 1).start()
            vdesc(b0 + 3, 1).start()

    plsc.subcore_barrier()
    pltpu.sync_copy(acc.at[pl.ds(row0, ROWS_PER_TILE)],
                    y_hbm.at[pl.ds(coff + row0, ROWS_PER_TILE)])


def _spmm(x_st, cols4, rows4, vals4):
    mesh = plsc.VectorSubcoreMesh(core_axis_name="c", subcore_axis_name="s",
                                  num_cores=NC, num_subcores=NS)
    f = pl.kernel(
        _spmm_body,
        out_type=jax.ShapeDtypeStruct((2 * NP, WHP), jnp.float32),
        mesh=mesh,
        scratch_types=[
            pltpu.VMEM_SHARED((NP, WHP), jnp.float32),
            pltpu.VMEM((2, 1, B), jnp.int32),
            pltpu.VMEM((2, 1, B), jnp.int32),
            pltpu.VMEM((2, 1, B), jnp.float32),
            pltpu.VMEM((2, B, WHP), jnp.float32),
        ] + [pltpu.SemaphoreType.DMA] * 10,
    )
    return f(x_st, cols4, rows4, vals4)


_R = 512


def _tc1(x, W0, b0, W1, b1, W2, b2, interpret=False):
    def body(x_ref, W0_ref, b0_ref, W1_ref, b1_ref, W2_ref, b2_ref,
             t0_ref, t1_ref, t2_ref):
        xb = x_ref[...]
        for W_ref, b_ref, t_ref in ((W0_ref, b0_ref, t0_ref),
                                    (W1_ref, b1_ref, t1_ref),
                                    (W2_ref, b2_ref, t2_ref)):
            acc = jnp.dot(xb, W_ref[0], preferred_element_type=jnp.float32)
            t_ref[...] = jnp.maximum(acc + b_ref[0], 0.0)

    nrb = NP // _R
    wspec = pl.BlockSpec((1, D, WHP), lambda i: (i // nrb, 0, 0))
    bspec = pl.BlockSpec((1, 1, WHP), lambda i: (i // nrb, 0, 0))
    ospec = pl.BlockSpec((_R, WHP), lambda i: (i, 0))
    oshape = jax.ShapeDtypeStruct((2 * NP, WHP), jnp.float32)
    return pl.pallas_call(
        body,
        grid=(2 * nrb,),
        in_specs=[pl.BlockSpec((_R, D), lambda i: (i % nrb, 0)),
                  wspec, bspec, wspec, bspec, wspec, bspec],
        out_specs=[ospec, ospec, ospec],
        out_shape=[oshape, oshape, oshape],
        interpret=interpret,
    )(x, W0, b0, W1, b1, W2, b2)


def _tc2(a1, Wg, bg, Wp1, Wp2, interpret=False):
    K = a1.shape[1]

    def body(a_ref, Wg_ref, bg_ref, W1_ref, W2_ref, g_ref, p1_ref, p2_ref):
        ab = a_ref[...]
        g_ref[...] = jnp.dot(ab, Wg_ref[0],
                             preferred_element_type=jnp.float32) + bg_ref[0]
        p1_ref[...] = jnp.dot(ab, W1_ref[0], preferred_element_type=jnp.float32)
        p2_ref[...] = jnp.dot(ab, W2_ref[0], preferred_element_type=jnp.float32)

    nrb = NP // _R
    wspec = pl.BlockSpec((1, K, WHP), lambda i: (i // nrb, 0, 0))
    bspec = pl.BlockSpec((1, 1, WHP), lambda i: (i // nrb, 0, 0))
    ospec = pl.BlockSpec((_R, WHP), lambda i: (i, 0))
    oshape = jax.ShapeDtypeStruct((2 * NP, WHP), jnp.float32)
    return pl.pallas_call(
        body,
        grid=(2 * nrb,),
        in_specs=[pl.BlockSpec((_R, K), lambda i: (i % nrb, 0)),
                  wspec, bspec, wspec, wspec],
        out_specs=[ospec, ospec, ospec],
        out_shape=[oshape, oshape, oshape],
        interpret=interpret,
    )(a1, Wg, bg, Wp1, Wp2)


def _tc3(a2, b2c, fcW, fcb, interpret=False):
    K = a2.shape[1]
    C = fcW.shape[1]

    def body(a_ref, b2_ref, W_ref, fb_ref, o_ref):
        ab = a_ref[...] + b2_ref[...]
        lg = jnp.dot(ab, W_ref[...],
                     preferred_element_type=jnp.float32) + fb_ref[...]
        colid = lax.broadcasted_iota(jnp.int32, (_R, C), 1)
        lg = jnp.where(colid < 40, lg, -1e30)
        m = jnp.max(lg, axis=1, keepdims=True)
        ssum = jnp.sum(jnp.exp(lg - m), axis=1, keepdims=True)
        out = lg - m - jnp.log(ssum)
        o_ref[...] = out[:, :40]

    nrb = NP // _R
    return pl.pallas_call(
        body,
        grid=(nrb,),
        in_specs=[pl.BlockSpec((_R, K), lambda i: (i, 0)),
                  pl.BlockSpec((1, K), lambda i: (0, 0)),
                  pl.BlockSpec((K, C), lambda i: (0, 0)),
                  pl.BlockSpec((1, C), lambda i: (0, 0))],
        out_specs=pl.BlockSpec((_R, 40), lambda i: (i, 0)),
        out_shape=jax.ShapeDtypeStruct((NP, 40), jnp.float32),
        interpret=interpret,
    )(a2, b2c, fcW, fcb)


def _halves(W):
    Wr = W.reshape(W.shape[0], 2, 100)
    Wr = jnp.pad(Wr, ((0, 0), (0, 0), (0, WHP - 100)))
    return Wr.transpose(1, 0, 2)


def _expand6(W):
    Wr = W.reshape(6, 100, W.shape[1])
    Wr = jnp.pad(Wr, ((0, 0), (0, WHP - 100), (0, 0)))
    return Wr.reshape(6 * WHP, W.shape[1])


def _unstack(t_st):
    return [t_st[:NP], t_st[NP:]]


def kernel(x, adj_indices, adj_values, W1_0, b1_0, W1_1, b1_1, W1_2, b1_2,
           W2_0, b2_0, W2_1, b2_1, W2_2, b2_2, fc_W, fc_b):
    rows4 = jnp.pad(adj_indices[0],
                    (0, NTOT - NNZ)).reshape(NS, EB_TILE, 1, B)
    cols4 = jnp.pad(adj_indices[1],
                    (0, NTOT - NNZ)).reshape(NS, EB_TILE, 1, B)
    vals4 = jnp.pad(adj_values, (0, NTOT - NNZ)).reshape(NS, EB_TILE, 1, B)

    xp = jnp.pad(x, ((0, NP - N), (0, 0)))
    t0_st, t1_st, t2_st = _tc1(xp, _halves(W1_0), _halves(b1_0),
                               _halves(W1_1), _halves(b1_1),
                               _halves(W1_2), _halves(b1_2))
    h1_st = _spmm(t1_st, cols4, rows4, vals4)
    u2_st = _spmm(t2_st, cols4, rows4, vals4)
    h2_st = _spmm(u2_st, cols4, rows4, vals4)

    a1 = jnp.concatenate(_unstack(t0_st) + _unstack(h1_st) + _unstack(h2_st),
                         axis=1)

    g0_st, p1_st, p2_st = _tc2(a1, _halves(_expand6(W2_0)), _halves(b2_0),
                               _halves(_expand6(W2_1)),
                               _halves(_expand6(W2_2)))
    v1_st = _spmm(p1_st, cols4, rows4, vals4)
    v2_st = _spmm(p2_st, cols4, rows4, vals4)
    w2_st = _spmm(v2_st, cols4, rows4, vals4)

    a2 = jnp.concatenate(_unstack(g0_st) + _unstack(v1_st) + _unstack(w2_st),
                         axis=1)
    b2c = jnp.concatenate([jnp.zeros((1, 2 * WHP), jnp.float32),
                           _halves(b2_1).reshape(1, 2 * WHP),
                           _halves(b2_2).reshape(1, 2 * WHP)], axis=1)
    fcW = jnp.pad(_expand6(fc_W), ((0, 0), (0, 88)))
    fcb = jnp.pad(fc_b.reshape(1, 40), ((0, 0), (0, 88)))
    return _tc3(a2, b2c, fcW, fcb)[:N]


# speedup vs baseline: 1.0118x; 1.0118x over previous
import functools

import jax
import jax.numpy as jnp
from jax import lax
from jax.experimental import pallas as pl
from jax.experimental.pallas import tpu as pltpu
from jax.experimental.pallas import tpu_sc as plsc

N = 10000
NP = 10240
D = 128
NNZ = 330000
NC = 2
NS = 16
WHP = 128
B = 128
ROWS_PER_TILE = NP // NS
EB_TILE = -(-NNZ // (NS * B))
E_TILE = EB_TILE * B
NTOT = E_TILE * NS
NK = WHP // 16
NKS = 7


def _lane_bcast(vv, e):
    return lax.gather(
        vv, jnp.full((16, 1), e, jnp.int32),
        lax.GatherDimensionNumbers(offset_dims=(), collapsed_slice_dims=(0,),
                                   start_index_map=(0,)),
        (1,), mode=lax.GatherScatterMode.PROMISE_IN_BOUNDS)


def _spmm_body(x_hbm, cols_hbm, rows_hbm, vals_hbm, y_hbm,
               acc, cbuf, rbuf, vbuf, gath, gsem0, gsem1, csem0, csem1,
               rsem0, rsem1, vsem0, vsem1, ssem0, ssem1):
    c = lax.axis_index("c")
    s = lax.axis_index("s")
    row0 = s * ROWS_PER_TILE
    coff = c * NP
    gsems = (gsem0, gsem1)
    csems = (csem0, csem1)
    rsems = (rsem0, rsem1)
    vsems = (vsem0, vsem1)
    ssems = (ssem0, ssem1)
    NT = EB_TILE // 2

    zero16 = jnp.zeros((16,), jnp.float32)

    @pl.loop(0, B)
    def _zero_gath(j):
        for k in range(NK):
            gath[0, j, pl.ds(k * 16, 16)] = zero16

    @pl.loop(0, ROWS_PER_TILE // B)
    def _zero_acc(i):
        pltpu.sync_copy(gath.at[0], acc.at[pl.ds(row0 + i * B, B)])

    plsc.subcore_barrier()

    def cdesc(b, u):
        return pltpu.make_async_copy(cols_hbm.at[s, b], cbuf.at[u], csems[u])

    def rdesc(b, u):
        return pltpu.make_async_copy(rows_hbm.at[s, b], rbuf.at[u], rsems[u])

    def vdesc(b, u):
        return pltpu.make_async_copy(vals_hbm.at[s, b], vbuf.at[u], vsems[u])

    def gdesc(u):
        return pltpu.make_async_copy(x_hbm.at[cbuf.at[u, 0]], gath.at[u],
                                     gsems[u])

    def add_coff(u):
        for k in range(B // 16):
            sl = (u, 0, pl.ds(k * 16, 16))
            cbuf[sl] = cbuf[sl] + coff

    def sdesc(u):
        return pltpu.make_async_copy(gath.at[u], acc.at[rbuf.at[u, 0]],
                                     ssems[u])

    def scale(u):
        @pl.loop(0, B // 16, unroll=2)
        def _scale(g):
            vv = vbuf[u, 0, pl.ds(g * 16, 16)]
            for e in range(16):
                v = _lane_bcast(vv, e)
                j = g * 16 + e
                for k in range(NKS):
                    sl = (u, j, pl.ds(k * 16, 16))
                    gath[sl] = gath[sl] * v

    pltpu.sync_copy(cols_hbm.at[s, 0], cbuf.at[0])
    rdesc(0, 0).start()
    vdesc(0, 0).start()
    add_coff(0)
    gdesc(0).start()
    cdesc(1, 1).start()
    rdesc(1, 1).start()
    vdesc(1, 1).start()

    def process(b, u):
        rdesc(b, u).wait()
        vdesc(b, u).wait()
        scale(u)
        pltpu.sync_copy(gath.at[u], acc.at[rbuf.at[u, 0]], add=True)

    @pl.loop(0, NT)
    def _pair(t):
        b0 = t * 2
        not_last = t < NT - 1

        cdesc(b0 + 1, 1).wait()
        add_coff(1)
        gdesc(1).start()
        gdesc(0).wait()

        @pl.when(not_last)
        def _():
            cdesc(b0 + 2, 0).start()
        process(b0, 0)

        @pl.when(not_last)
        def _():
            rdesc(b0 + 2, 0).start()
            vdesc(b0 + 2, 0).start()

        @pl.when(not_last)
        def _():
            cdesc(b0 + 2, 0).wait()
            add_coff(0)
            gdesc(0).start()
        gdesc(1).wait()

        process(b0 + 1, 1)

        @pl.when(not_last)
        def _():
            cdesc(b0 + 3, 1).start()
            rdesc(b0 + 3, 1).start()
            vdesc(b0 + 3, 1).start()

    plsc.subcore_barrier()
    pltpu.sync_copy(acc.at[pl.ds(row0, ROWS_PER_TILE)],
                    y_hbm.at[pl.ds(coff + row0, ROWS_PER_TILE)])


def _spmm(x_st, cols4, rows4, vals4):
    mesh = plsc.VectorSubcoreMesh(core_axis_name="c", subcore_axis_name="s",
                                  num_cores=NC, num_subcores=NS)
    f = pl.kernel(
        _spmm_body,
        out_type=jax.ShapeDtypeStruct((2 * NP, WHP), jnp.float32),
        mesh=mesh,
        scratch_types=[
            pltpu.VMEM_SHARED((NP, WHP), jnp.float32),
            pltpu.VMEM((2, 1, B), jnp.int32),
            pltpu.VMEM((2, 1, B), jnp.int32),
            pltpu.VMEM((2, 1, B), jnp.float32),
            pltpu.VMEM((2, B, WHP), jnp.float32),
        ] + [pltpu.SemaphoreType.DMA] * 10,
    )
    return f(x_st, cols4, rows4, vals4)


_R = 512


def _tc1(x, W0, b0, W1, b1, W2, b2, interpret=False):
    def body(x_ref, W0_ref, b0_ref, W1_ref, b1_ref, W2_ref, b2_ref,
             t0_ref, t1_ref, t2_ref):
        xb = x_ref[...]
        for W_ref, b_ref, t_ref in ((W0_ref, b0_ref, t0_ref),
                                    (W1_ref, b1_ref, t1_ref),
                                    (W2_ref, b2_ref, t2_ref)):
            acc = jnp.dot(xb, W_ref[0], preferred_element_type=jnp.float32)
            t_ref[...] = jnp.maximum(acc + b_ref[0], 0.0)

    nrb = NP // _R
    wspec = pl.BlockSpec((1, D, WHP), lambda i: (i // nrb, 0, 0))
    bspec = pl.BlockSpec((1, 1, WHP), lambda i: (i // nrb, 0, 0))
    ospec = pl.BlockSpec((_R, WHP), lambda i: (i, 0))
    oshape = jax.ShapeDtypeStruct((2 * NP, WHP), jnp.float32)
    return pl.pallas_call(
        body,
        grid=(2 * nrb,),
        in_specs=[pl.BlockSpec((_R, D), lambda i: (i % nrb, 0)),
                  wspec, bspec, wspec, bspec, wspec, bspec],
        out_specs=[ospec, ospec, ospec],
        out_shape=[oshape, oshape, oshape],
        interpret=interpret,
    )(x, W0, b0, W1, b1, W2, b2)


def _tc2(a1, Wg, bg, Wp1, Wp2, interpret=False):
    K = a1.shape[1]

    def body(a_ref, Wg_ref, bg_ref, W1_ref, W2_ref, g_ref, p1_ref, p2_ref):
        ab = a_ref[...]
        g_ref[...] = jnp.dot(ab, Wg_ref[0],
                             preferred_element_type=jnp.float32) + bg_ref[0]
        p1_ref[...] = jnp.dot(ab, W1_ref[0], preferred_element_type=jnp.float32)
        p2_ref[...] = jnp.dot(ab, W2_ref[0], preferred_element_type=jnp.float32)

    nrb = NP // _R
    wspec = pl.BlockSpec((1, K, WHP), lambda i: (i // nrb, 0, 0))
    bspec = pl.BlockSpec((1, 1, WHP), lambda i: (i // nrb, 0, 0))
    ospec = pl.BlockSpec((_R, WHP), lambda i: (i, 0))
    oshape = jax.ShapeDtypeStruct((2 * NP, WHP), jnp.float32)
    return pl.pallas_call(
        body,
        grid=(2 * nrb,),
        in_specs=[pl.BlockSpec((_R, K), lambda i: (i % nrb, 0)),
                  wspec, bspec, wspec, wspec],
        out_specs=[ospec, ospec, ospec],
        out_shape=[oshape, oshape, oshape],
        interpret=interpret,
    )(a1, Wg, bg, Wp1, Wp2)


def _tc3(a2, b2c, fcW, fcb, interpret=False):
    K = a2.shape[1]
    C = fcW.shape[1]

    def body(a_ref, b2_ref, W_ref, fb_ref, o_ref):
        ab = a_ref[...] + b2_ref[...]
        lg = jnp.dot(ab, W_ref[...],
                     preferred_element_type=jnp.float32) + fb_ref[...]
        colid = lax.broadcasted_iota(jnp.int32, (_R, C), 1)
        lg = jnp.where(colid < 40, lg, -1e30)
        m = jnp.max(lg, axis=1, keepdims=True)
        ssum = jnp.sum(jnp.exp(lg - m), axis=1, keepdims=True)
        out = lg - m - jnp.log(ssum)
        o_ref[...] = out[:, :40]

    nrb = NP // _R
    return pl.pallas_call(
        body,
        grid=(nrb,),
        in_specs=[pl.BlockSpec((_R, K), lambda i: (i, 0)),
                  pl.BlockSpec((1, K), lambda i: (0, 0)),
                  pl.BlockSpec((K, C), lambda i: (0, 0)),
                  pl.BlockSpec((1, C), lambda i: (0, 0))],
        out_specs=pl.BlockSpec((_R, 40), lambda i: (i, 0)),
        out_shape=jax.ShapeDtypeStruct((NP, 40), jnp.float32),
        interpret=interpret,
    )(a2, b2c, fcW, fcb)


def _halves(W):
    Wr = W.reshape(W.shape[0], 2, 100)
    Wr = jnp.pad(Wr, ((0, 0), (0, 0), (0, WHP - 100)))
    return Wr.transpose(1, 0, 2)


def _expand6(W):
    Wr = W.reshape(6, 100, W.shape[1])
    Wr = jnp.pad(Wr, ((0, 0), (0, WHP - 100), (0, 0)))
    return Wr.reshape(6 * WHP, W.shape[1])


def _unstack(t_st):
    return [t_st[:NP], t_st[NP:]]


def kernel(x, adj_indices, adj_values, W1_0, b1_0, W1_1, b1_1, W1_2, b1_2,
           W2_0, b2_0, W2_1, b2_1, W2_2, b2_2, fc_W, fc_b):
    rows4 = jnp.pad(adj_indices[0],
                    (0, NTOT - NNZ)).reshape(NS, EB_TILE, 1, B)
    cols4 = jnp.pad(adj_indices[1],
                    (0, NTOT - NNZ)).reshape(NS, EB_TILE, 1, B)
    vals4 = jnp.pad(adj_values, (0, NTOT - NNZ)).reshape(NS, EB_TILE, 1, B)

    xp = jnp.pad(x, ((0, NP - N), (0, 0)))
    t0_st, t1_st, t2_st = _tc1(xp, _halves(W1_0), _halves(b1_0),
                               _halves(W1_1), _halves(b1_1),
                               _halves(W1_2), _halves(b1_2))
    h1_st = _spmm(t1_st, cols4, rows4, vals4)
    u2_st = _spmm(t2_st, cols4, rows4, vals4)
    h2_st = _spmm(u2_st, cols4, rows4, vals4)

    a1 = jnp.concatenate(_unstack(t0_st) + _unstack(h1_st) + _unstack(h2_st),
                         axis=1)

    g0_st, p1_st, p2_st = _tc2(a1, _halves(_expand6(W2_0)), _halves(b2_0),
                               _halves(_expand6(W2_1)),
                               _halves(_expand6(W2_2)))
    v1_st = _spmm(p1_st, cols4, rows4, vals4)
    v2_st = _spmm(p2_st, cols4, rows4, vals4)
    w2_st = _spmm(v2_st, cols4, rows4, vals4)

    a2 = jnp.concatenate(_unstack(g0_st) + _unstack(v1_st) + _unstack(w2_st),
                         axis=1)
    b2c = jnp.concatenate([jnp.zeros((1, 2 * WHP), jnp.float32),
                           _halves(b2_1).reshape(1, 2 * WHP),
                           _halves(b2_2).reshape(1, 2 * WHP)], axis=1)
    fcW = jnp.pad(_expand6(fc_W), ((0, 0), (0, 88)))
    fcb = jnp.pad(fc_b.reshape(1, 40), ((0, 0), (0, 88)))
    return _tc3(a2, b2c, fcW, fcb)[:N]
